# baseline (speedup 1.0000x reference)
import jax
import jax.numpy as jnp
from jax import lax
from jax.experimental import pallas as pl
from jax.experimental.pallas import tpu as pltpu


def _body(col_ref, row_ref, out_ref, pat_ref):
    w, d2 = col_ref.shape
    h = row_ref.shape[0]
    hw = h * w

    @pl.when(pl.program_id(0) == 0)
    def _():
        lane = lax.broadcasted_iota(jnp.int32, (w, hw), 1)
        sub = lax.broadcasted_iota(jnp.int32, (w, hw), 0)
        sel_col = (lane % w == sub).astype(jnp.float32)
        sel_row = (lane // w == sub).astype(jnp.float32)
        dn = (((0,), (0,)), ((), ()))
        a = lax.dot_general(col_ref[...], sel_col, dn,
                            preferred_element_type=jnp.float32,
                            precision=lax.Precision.HIGHEST)
        bb = lax.dot_general(row_ref[...], sel_row, dn,
                             preferred_element_type=jnp.float32,
                             precision=lax.Precision.HIGHEST)
        pat_ref[...] = jnp.concatenate([a, bb], axis=0)

    out_ref[0] = pat_ref[...]


def kernel(tensor, row_embed, col_embed):
    b = tensor.shape[0]
    h, w = tensor.shape[-2], tensor.shape[-1]
    d2 = row_embed.shape[-1]
    d = 2 * d2
    out = pl.pallas_call(
        _body,
        grid=(b,),
        in_specs=[
            pl.BlockSpec((w, d2), lambda i: (0, 0)),
            pl.BlockSpec((h, d2), lambda i: (0, 0)),
        ],
        out_specs=pl.BlockSpec((1, d, h * w), lambda i: (i, 0, 0)),
        out_shape=jax.ShapeDtypeStruct((b, d, h * w), jnp.float32),
        scratch_shapes=[
            pltpu.VMEM((d, h * w), jnp.float32),
        ],
    )(col_embed, row_embed)
    return out.reshape(b, d, h, w)
